# baseline (speedup 1.0000x reference)
import functools

import jax
import jax.numpy as jnp
from jax import lax
from jax.experimental import pallas as pl
from jax.experimental.pallas import tpu as pltpu
from jax.experimental.pallas import tpu_sc as plsc

N = 100000
E = 3200000
U = 64
EPS = 1e-5

NW = 32
NTEC = 16
NB = 704
CB = 144
CPAD = 160
DUMP = 144
SHIFT = 8
VMASK = 255
CAP = 320
TOT = NB * NW * CAP
NP = NB * CB
EPT = E // NW
CHUNK = 2000
NCHUNK = EPT // CHUNK
VPC = CHUNK // 16
K2 = 256
K = 256
EBMAX = 6144
BPW = NB // NW
U2 = 128
FB = 144
RB = 1024
NRB = NP // RB

_MESH_CACHE = []


def _mesh():
    if not _MESH_CACHE:
        _MESH_CACHE.append(
            plsc.VectorSubcoreMesh(core_axis_name="c", subcore_axis_name="s"))
    return _MESH_CACHE[0]


def _bucket_of(d):
    d16 = lax.shift_right_logical(d, 4)
    return lax.shift_right_logical(d16 * 29128, 18)


def _producer_body(src_hbm, dst_hbm, barr_hbm, cnt_hbm,
                   sbuf, dbuf, vstage, pstage, cur, dumpb, posb, sem):
    wid = lax.axis_index("c") * NTEC + lax.axis_index("s")
    iot = lax.iota(jnp.int32, 16)

    for i in range(NB // 16):
        cur[pl.ds(i * 16, 16)] = jnp.zeros((16,), jnp.int32)
        posb[pl.ds(i * 16, 16)] = (iot + i * 16) * NW + wid
    for i in range(CAP // 16):
        dumpb[pl.ds(i * 16, 16)] = jnp.full((16,), DUMP, jnp.int32)

    def initb(k, carry):
        soff = pl.multiple_of(wid * (NB * CAP) + k * CAP, 64)
        pltpu.sync_copy(dumpb, barr_hbm.at[pl.ds(soff, CAP)])
        return carry

    lax.fori_loop(0, NB, initb, 0)

    ebase = wid * EPT

    def chunk_body(ch, carry):
        off = pl.multiple_of(ebase + ch * CHUNK, 16)
        pltpu.sync_copy(src_hbm.at[pl.ds(off, CHUNK)], sbuf)
        pltpu.sync_copy(dst_hbm.at[pl.ds(off, CHUNK)], dbuf)

        def vec_body(v, c2):
            s16 = sbuf[pl.ds(pl.multiple_of(v * 16, 16), 16)]
            d16 = dbuf[pl.ds(pl.multiple_of(v * 16, 16), 16)]
            b = _bucket_of(d16)
            dl = d16 - b * CB
            val = lax.shift_left(s16, SHIFT) | dl
            ks, lane_s = plsc.sort_key_val(b, iot)
            prev = ks.at[jnp.maximum(iot - 1, 0)].get(mode="promise_in_bounds")
            segstart = jnp.where((iot == 0) | (ks != prev), iot, 0)
            rank = iot - plsc.cummax(segstart)
            nxt = ks.at[jnp.minimum(iot + 1, 15)].get(mode="promise_in_bounds")
            end = (iot == 15) | (ks != nxt)
            c_s = plsc.load_gather(cur, [ks])
            plsc.store_scatter(cur, [ks], jnp.minimum(c_s + rank + 1, CAP),
                               mask=end)
            pos = (wid * NB + ks) * CAP + jnp.minimum(c_s + rank, CAP - 1)
            val_s = val.at[lane_s].get(mode="promise_in_bounds")
            vstage[pl.ds(pl.multiple_of(v * 16, 16), 16)] = val_s
            pstage[pl.ds(pl.multiple_of(v * 16, 16), 16)] = pos
            return c2

        lax.fori_loop(0, VPC, vec_body, 0)
        pltpu.async_copy(vstage, barr_hbm.at[pstage], sem).wait()
        return carry

    lax.fori_loop(0, NCHUNK, chunk_body, 0)
    pltpu.async_copy(cur, cnt_hbm.at[posb], sem).wait()


def _producer(src, dst):
    return pl.kernel(
        _producer_body,
        out_type=(jax.ShapeDtypeStruct((TOT,), jnp.int32),
                  jax.ShapeDtypeStruct((NB * NW,), jnp.int32)),
        mesh=_mesh(),
        compiler_params=pltpu.CompilerParams(needs_layout_passes=False),
        scratch_types=[
            pltpu.VMEM((CHUNK,), jnp.int32),
            pltpu.VMEM((CHUNK,), jnp.int32),
            pltpu.VMEM((CHUNK,), jnp.int32),
            pltpu.VMEM((CHUNK,), jnp.int32),
            pltpu.VMEM((NB,), jnp.int32),
            pltpu.VMEM((CAP,), jnp.int32),
            pltpu.VMEM((NB,), jnp.int32),
            pltpu.SemaphoreType.DMA,
        ],
    )(src, dst)


def _assemble_bucket(barr_hbm, cnt_hbm, cntv, ebuf, b):
    pltpu.sync_copy(cnt_hbm.at[pl.ds(pl.multiple_of(b * NW, 32), NW)], cntv)
    cv = [cntv[pl.ds(0, 16)], cntv[pl.ds(16, 16)]]
    epos = jnp.int32(0)
    for t in range(NW):
        c = jnp.minimum(cv[t // 16][t % 16], CAP)
        trips = lax.shift_right_logical(c + (K2 - 1), 8)
        sbase = pl.multiple_of((t * NB + b) * CAP, 64)
        epos = pl.multiple_of(epos, 8)

        def cp(j, c2, sbase=sbase, epos=epos):
            pltpu.sync_copy(
                barr_hbm.at[pl.ds(pl.multiple_of(sbase + j * K2, 8), K2)],
                ebuf.at[pl.ds(pl.multiple_of(epos + j * K2, 8), K2)])
            return c2

        lax.fori_loop(0, trips, cp, 0)
        epos = epos + lax.shift_left(lax.shift_right_logical(c + 7, 3), 3)

    etot = pl.multiple_of(jnp.minimum(epos, EBMAX - K), 8)
    dumpvec = jnp.full((16,), DUMP, jnp.int32)
    for i in range(K // 16):
        ebuf[pl.ds(pl.multiple_of(etot + i * 16, 8), 16)] = dumpvec
    return lax.shift_right_logical(etot + (K - 1), 8)


def _agg64_body(g_hbm, barr_hbm, cnt_hbm, s_hbm,
                cntv, ebuf, six, dlx, rows, zbuf, fbuf, acc_sh, gsem, asem):
    sid = lax.axis_index("s")
    wid = lax.axis_index("c") * NTEC + sid
    accbase = sid * CPAD

    for i in range(16):
        for q in range(U2 // 16):
            zbuf[i, pl.ds(q * 16, 16)] = jnp.zeros((16,), jnp.float32)

    def bucket_body(bi, bcarry):
        b = wid * BPW + bi

        def zr(r, carry):
            pltpu.sync_copy(zbuf, acc_sh.at[pl.ds(pl.multiple_of(accbase + r * 16, 16), 16)])
            return carry

        lax.fori_loop(0, CPAD // 16, zr, 0)

        ptot = _assemble_bucket(barr_hbm, cnt_hbm, cntv, ebuf, b)

        def proc(j, carry):
            for q in range(K // 16):
                v = ebuf[pl.ds(pl.multiple_of(j * K + q * 16, 16), 16)]
                six[pl.ds(q * 16, 16)] = lax.shift_right_logical(v, SHIFT)
                dlx[pl.ds(q * 16, 16)] = (v & VMASK) + accbase
            pltpu.async_copy(g_hbm.at[six], rows, gsem).wait()
            pltpu.async_copy(rows, acc_sh.at[dlx], asem, add=True).wait()
            return carry

        lax.fori_loop(0, ptot, proc, 0)

        pltpu.sync_copy(acc_sh.at[pl.ds(accbase, FB)], fbuf)
        pltpu.sync_copy(fbuf, s_hbm.at[pl.ds(pl.multiple_of(b * CB, 8), FB)])
        return bcarry

    lax.fori_loop(0, BPW, bucket_body, 0)


def _agg64(g, barr, cnt):
    return pl.kernel(
        _agg64_body,
        out_type=jax.ShapeDtypeStruct((NP, U2), jnp.float32),
        mesh=_mesh(),
        compiler_params=pltpu.CompilerParams(needs_layout_passes=False),
        scratch_types=[
            pltpu.VMEM((NW,), jnp.int32),
            pltpu.VMEM((EBMAX,), jnp.int32),
            pltpu.VMEM((K,), jnp.int32),
            pltpu.VMEM((K,), jnp.int32),
            pltpu.VMEM((K, U2), jnp.float32),
            pltpu.VMEM((16, U2), jnp.float32),
            pltpu.VMEM((FB, U2), jnp.float32),
            pltpu.VMEM_SHARED((NTEC * CPAD, U2), jnp.float32),
            pltpu.SemaphoreType.DMA,
            pltpu.SemaphoreType.DMA,
        ],
    )(g, barr, cnt)


def _make_agg1(count_mode):
    def body(g_hbm, barr_hbm, cnt_hbm, s_hbm,
             cntv, ebuf, six, dlx, rows, zbuf, fbuf, acc_sh, gsem, asem):
        sid = lax.axis_index("s")
        wid = lax.axis_index("c") * NTEC + sid
        accbase = sid * CPAD

        zbuf[...] = jnp.zeros((16,), jnp.float32)
        if count_mode:
            for i in range(K // 16):
                rows[pl.ds(i * 16, 16)] = jnp.ones((16,), jnp.float32)

        def bucket_body(bi, bcarry):
            b = wid * BPW + bi

            def zr(r, carry):
                pltpu.sync_copy(zbuf, acc_sh.at[pl.ds(pl.multiple_of(accbase + r * 16, 16), 16)])
                return carry

            lax.fori_loop(0, CPAD // 16, zr, 0)

            ptot = _assemble_bucket(barr_hbm, cnt_hbm, cntv, ebuf, b)

            def proc(j, carry):
                for q in range(K // 16):
                    v = ebuf[pl.ds(pl.multiple_of(j * K + q * 16, 16), 16)]
                    if not count_mode:
                        six[pl.ds(q * 16, 16)] = lax.shift_right_logical(
                            v, SHIFT)
                    dlx[pl.ds(q * 16, 16)] = (v & VMASK) + accbase
                if not count_mode:
                    pltpu.async_copy(g_hbm.at[six], rows, gsem).wait()
                pltpu.async_copy(rows, acc_sh.at[dlx], asem, add=True).wait()
                return carry

            lax.fori_loop(0, ptot, proc, 0)

            pltpu.sync_copy(acc_sh.at[pl.ds(accbase, CPAD)], fbuf)
            pltpu.sync_copy(fbuf,
                            s_hbm.at[pl.ds(pl.multiple_of(b * CPAD, 32),
                                           CPAD)])
            return bcarry

        lax.fori_loop(0, BPW, bucket_body, 0)

    def run(g, barr, cnt):
        return pl.kernel(
            body,
            out_type=jax.ShapeDtypeStruct((NB * CPAD,), jnp.float32),
            mesh=_mesh(),
        compiler_params=pltpu.CompilerParams(needs_layout_passes=False),
            scratch_types=[
                pltpu.VMEM((NW,), jnp.int32),
                pltpu.VMEM((EBMAX,), jnp.int32),
                pltpu.VMEM((K,), jnp.int32),
                pltpu.VMEM((K,), jnp.int32),
                pltpu.VMEM((K,), jnp.float32),
                pltpu.VMEM((16,), jnp.float32),
                pltpu.VMEM((CPAD,), jnp.float32),
                pltpu.VMEM_SHARED((NTEC * CPAD,), jnp.float32),
                pltpu.SemaphoreType.DMA,
                pltpu.SemaphoreType.DMA,
            ],
        )(g, barr, cnt)

    return run


def _dot(a, b):
    return lax.dot_general(a, b, (((1,), (0,)), ((), ())),
                           precision=lax.Precision.HIGHEST,
                           preferred_element_type=jnp.float32)


def _tc_prep_body(cnt_ref, x_ref, dinv_ref, g0_ref):
    d = lax.rsqrt(cnt_ref[...] + 1.0)
    dinv_ref[...] = d
    g0_ref[...] = d * x_ref[...]


def _tc_prep(degc, xp):
    return pl.pallas_call(
        _tc_prep_body,
        grid=(NRB,),
        in_specs=[pl.BlockSpec((RB, 1), lambda i: (i, 0)),
                  pl.BlockSpec((RB, 1), lambda i: (i, 0))],
        out_specs=[pl.BlockSpec((RB, 1), lambda i: (i, 0)),
                   pl.BlockSpec((RB, 1), lambda i: (i, 0))],
        out_shape=[jax.ShapeDtypeStruct((NP, 1), jnp.float32),
                   jax.ShapeDtypeStruct((NP, 1), jnp.float32)],
    )(degc, xp)


def _stats_update(st_ref, z, i):
    sz = jnp.sum(z, axis=0, keepdims=True)
    szz = jnp.sum(z * z, axis=0, keepdims=True)
    st = jnp.concatenate([sz, szz], axis=0)

    @pl.when(i == 0)
    def _():
        st_ref[...] = st

    @pl.when(i > 0)
    def _():
        st_ref[...] = st_ref[...] + st


def _tc_l0_body(s_ref, g_ref, dinv_ref, w0_ref, b0_ref, z_ref, st_ref):
    i = pl.program_id(0)
    u = dinv_ref[...] * (s_ref[...] + g_ref[...])
    z = u * w0_ref[...] + b0_ref[...]
    rows = lax.broadcasted_iota(jnp.int32, (RB, U), 0) + i * RB
    z = jnp.where(rows < N, z, 0.0)
    z_ref[...] = z
    _stats_update(st_ref, z, i)


def _tc_l0(s0, g0, dinv, w0, b0):
    return pl.pallas_call(
        _tc_l0_body,
        grid=(NRB,),
        in_specs=[pl.BlockSpec((RB, 1), lambda i: (i, 0)),
                  pl.BlockSpec((RB, 1), lambda i: (i, 0)),
                  pl.BlockSpec((RB, 1), lambda i: (i, 0)),
                  pl.BlockSpec((1, U), lambda i: (0, 0)),
                  pl.BlockSpec((1, U), lambda i: (0, 0))],
        out_specs=[pl.BlockSpec((RB, U), lambda i: (i, 0)),
                   pl.BlockSpec((2, U), lambda i: (0, 0))],
        out_shape=[jax.ShapeDtypeStruct((NP, U), jnp.float32),
                   jax.ShapeDtypeStruct((2, U), jnp.float32)],
    )(s0, g0, dinv, w0, b0)


def _tc_mid_body(s_ref, g_ref, dinv_ref, w_ref, b_ref, z_ref, st_ref):
    i = pl.program_id(0)
    u = dinv_ref[...] * (s_ref[...][:, :U] + g_ref[...][:, :U])
    z = _dot(u, w_ref[...]) + b_ref[...]
    rows = lax.broadcasted_iota(jnp.int32, (RB, U), 0) + i * RB
    z = jnp.where(rows < N, z, 0.0)
    z_ref[...] = z
    _stats_update(st_ref, z, i)


def _tc_mid(s, g, dinv, w, bb):
    return pl.pallas_call(
        _tc_mid_body,
        grid=(NRB,),
        in_specs=[pl.BlockSpec((RB, U2), lambda i: (i, 0)),
                  pl.BlockSpec((RB, U2), lambda i: (i, 0)),
                  pl.BlockSpec((RB, 1), lambda i: (i, 0)),
                  pl.BlockSpec((U, U), lambda i: (0, 0)),
                  pl.BlockSpec((1, U), lambda i: (0, 0))],
        out_specs=[pl.BlockSpec((RB, U), lambda i: (i, 0)),
                   pl.BlockSpec((2, U), lambda i: (0, 0))],
        out_shape=[jax.ShapeDtypeStruct((NP, U), jnp.float32),
                   jax.ShapeDtypeStruct((2, U), jnp.float32)],
    )(s, g, dinv, w, bb)


def _tc_norm_body(z_ref, st_ref, nw_ref, nb_ref, ms_ref, dinv_ref, g_ref):
    st = st_ref[...]
    mean = st[0:1, :] * (1.0 / N)
    ez2 = st[1:2, :] * (1.0 / N)
    ms = ms_ref[...]
    var = ez2 - (2.0 - ms) * ms * mean * mean
    scale = nw_ref[...] * lax.rsqrt(var + EPS)
    y = (z_ref[...] - ms * mean) * scale + nb_ref[...]
    h = y * jax.nn.sigmoid(y)
    g_ref[...] = jnp.concatenate(
        [dinv_ref[...] * h, jnp.zeros((RB, U2 - U), jnp.float32)], axis=1)


def _tc_norm(z, st, nw, nb, ms, dinv):
    return pl.pallas_call(
        _tc_norm_body,
        grid=(NRB,),
        in_specs=[pl.BlockSpec((RB, U), lambda i: (i, 0)),
                  pl.BlockSpec((2, U), lambda i: (0, 0)),
                  pl.BlockSpec((1, U), lambda i: (0, 0)),
                  pl.BlockSpec((1, U), lambda i: (0, 0)),
                  pl.BlockSpec((1, U), lambda i: (0, 0)),
                  pl.BlockSpec((RB, 1), lambda i: (i, 0))],
        out_specs=pl.BlockSpec((RB, U2), lambda i: (i, 0)),
        out_shape=jax.ShapeDtypeStruct((NP, U2), jnp.float32),
    )(z, st, nw, nb, ms, dinv)


def _tc_heads_body(s_ref, g_ref, dinv_ref, w11_ref, b11_ref,
                   pw_ref, pb_ref, pwo_ref, pbo_ref,
                   hw_ref, hb_ref, hwo_ref, hbo_ref,
                   phe_ref, heu_ref):
    u = dinv_ref[...] * (s_ref[...][:, :U] + g_ref[...][:, :U])
    hf = _dot(u, w11_ref[...]) + b11_ref[...]

    def head(ws, bs, wo, bo):
        t = hf
        for k in range(4):
            t = jnp.maximum(t + _dot(t, ws[k]) + bs[k:k + 1, :], 0.0)
        return jax.nn.sigmoid(_dot(t, wo) + bo)

    phe_ref[...] = head(pw_ref[...], pb_ref[...], pwo_ref[...], pbo_ref[...])
    heu_ref[...] = head(hw_ref[...], hb_ref[...], hwo_ref[...], hbo_ref[...])


def _tc_heads(s, g, dinv, w11, b11, pw, pb, pwo, pbo, hw, hb, hwo, hbo):
    full = lambda *dims: pl.BlockSpec(dims, lambda i: tuple(0 for _ in dims))
    return pl.pallas_call(
        _tc_heads_body,
        grid=(NRB,),
        in_specs=[pl.BlockSpec((RB, U2), lambda i: (i, 0)),
                  pl.BlockSpec((RB, U2), lambda i: (i, 0)),
                  pl.BlockSpec((RB, 1), lambda i: (i, 0)),
                  full(U, U), full(1, U),
                  full(4, U, U), full(4, U), full(U, 1), full(1, 1),
                  full(4, U, U), full(4, U), full(U, 1), full(1, 1)],
        out_specs=[pl.BlockSpec((RB, 1), lambda i: (i, 0)),
                   pl.BlockSpec((RB, 1), lambda i: (i, 0))],
        out_shape=[jax.ShapeDtypeStruct((NP, 1), jnp.float32),
                   jax.ShapeDtypeStruct((NP, 1), jnp.float32)],
    )(s, g, dinv, w11, b11, pw, pb, pwo, pbo, hw, hb, hwo, hbo)


_agg_w1 = _make_agg1(count_mode=False)
_agg_cnt = _make_agg1(count_mode=True)


def _unpad1(a):
    return a.reshape(NB, CPAD)[:, :CB].reshape(NP, 1)


def kernel(x, edge_index, convW0, convW, convB, normW, normB, normMS,
           pheW, pheB, pheWo, pheBo, heuW, heuB, heuWo, heuBo):
    xp = jnp.pad(x, ((0, NP - N), (0, 0)))
    src = edge_index[0]
    dst = edge_index[1]

    barr, cnt = _producer(src, dst)
    dummy = jnp.zeros((NP,), jnp.float32)
    degc = _agg_cnt(dummy, barr, cnt)
    dinv, g0 = _tc_prep(_unpad1(degc), xp)
    s0 = _agg_w1(g0.reshape(NP), barr, cnt)
    s0 = _unpad1(s0)
    z, st = _tc_l0(s0, g0, dinv, convW0,
                   convB[0].reshape(1, U))
    g = _tc_norm(z, st, normW[0].reshape(1, U), normB[0].reshape(1, U),
                 normMS[0].reshape(1, U), dinv)

    for i in range(1, 12):
        s = _agg64(g, barr, cnt)
        if i < 11:
            z, st = _tc_mid(s, g, dinv, convW[i - 1], convB[i].reshape(1, U))
            g = _tc_norm(z, st, normW[i].reshape(1, U),
                         normB[i].reshape(1, U), normMS[i].reshape(1, U),
                         dinv)
        else:
            phe2, heu2 = _tc_heads(
                s, g, dinv, convW[10], convB[11].reshape(1, U),
                pheW, pheB, pheWo, pheBo.reshape(1, 1),
                heuW, heuB, heuWo, heuBo.reshape(1, 1))

    return phe2.reshape(-1)[:N], heu2.reshape(-1)[:N]
